# Initial kernel scaffold; baseline (speedup 1.0000x reference)
import jax
import jax.numpy as jnp
from jax.experimental import pallas as pl


def kernel(img):
    raise NotImplementedError("write your pallas kernel here")


# speedup vs baseline: 9.8337x; 9.8337x over previous
import functools

import jax
import jax.numpy as jnp
from jax import lax
from jax.experimental import pallas as pl
from jax.experimental.pallas import tpu as pltpu
from jax.experimental.pallas import tpu_sc as plsc

B, C, H, W = 8, 3, 512, 512
N = B * H * W
NBINS = 16 * 4096

NC, NS, L = 2, 16, 16
NW = NC * NS
PER_TILE = N // NW
CHUNK = 8192
NCHUNKS = PER_TILE // CHUNK
UNROLL = 8


def _combo_body(img_ref, out_ref):
    x = img_ref[0]
    r, g, b = x[0], x[1], x[2]
    gray = (r * jnp.float32(0.299) + g * jnp.float32(0.587)) + b * jnp.float32(0.114)
    mean = (r + g + b) / jnp.float32(3.0)
    bits = lax.bitcast_convert_type(gray, jnp.uint32)
    combo = jnp.where(mean > jnp.float32(0.5), bits, bits | jnp.uint32(0x80000000))
    out_ref[0] = combo


def _make_combo(img):
    rows = 64
    grid = (B, H // rows)
    return pl.pallas_call(
        _combo_body,
        grid=grid,
        in_specs=[pl.BlockSpec((1, C, rows, W), lambda i, j: (i, 0, j, 0))],
        out_specs=pl.BlockSpec((1, rows, W), lambda i, j: (i, j, 0)),
        out_shape=jax.ShapeDtypeStruct((B, H, W), jnp.uint32),
    )(img)


def _zero_hist(hist_v, nwords):
    z = jnp.zeros((L,), jnp.int32)

    def zbody(i, _):
        for j in range(UNROLL):
            hist_v[pl.ds((i * UNROLL + j) * L, L)] = z
        return 0

    lax.fori_loop(0, nwords // (L * UNROLL), zbody, 0)


def _sc_hist_top_body(combo_hbm, out_hbm, stage_v, hist_v):
    wid = lax.axis_index("s") * NC + lax.axis_index("c")
    base = wid * PER_TILE
    _zero_hist(hist_v, NBINS)

    def chunk_body(ci, _):
        pltpu.sync_copy(combo_hbm.at[pl.ds(base + ci * CHUNK, CHUNK)], stage_v)

        def vec_body(vi, _):
            for j in range(UNROLL):
                k16 = stage_v[pl.ds((vi * UNROLL + j) * L, L)]
                bins = lax.convert_element_type(k16 >> 16, jnp.int32)
                cnts, last = plsc.scan_count(bins)
                plsc.addupdate_scatter(hist_v, [bins], cnts, mask=last)
            return 0

        lax.fori_loop(0, CHUNK // (L * UNROLL), vec_body, 0)
        return 0

    lax.fori_loop(0, NCHUNKS, chunk_body, 0)
    pltpu.sync_copy(hist_v, out_hbm.at[wid])


_JUNK = NBINS


def _sc_hist_low_body(combo_hbm, bsel_hbm, out_hbm, stage_v, hist_v, brow_v):
    wid = lax.axis_index("s") * NC + lax.axis_index("c")
    base = wid * PER_TILE
    _zero_hist(hist_v, NBINS + L)
    pltpu.sync_copy(bsel_hbm.at[0], brow_v)
    bv = brow_v[pl.ds(0, L)]
    junk = jnp.full((L,), _JUNK, jnp.int32)

    def chunk_body(ci, _):
        pltpu.sync_copy(combo_hbm.at[pl.ds(base + ci * CHUNK, CHUNK)], stage_v)

        def vec_body(vi, _):
            for j in range(UNROLL):
                k16 = stage_v[pl.ds((vi * UNROLL + j) * L, L)]
                top = lax.convert_element_type(k16 >> 16, jnp.int32)
                low = lax.convert_element_type(k16 & jnp.uint32(0xFFFF), jnp.int32)
                idx = jnp.where(top == bv, low, junk)
                cnts, last = plsc.scan_count(idx)
                plsc.addupdate_scatter(hist_v, [idx], cnts, mask=last)
            return 0

        lax.fori_loop(0, CHUNK // (L * UNROLL), vec_body, 0)
        return 0

    lax.fori_loop(0, NCHUNKS, chunk_body, 0)
    pltpu.sync_copy(hist_v.at[pl.ds(0, NBINS)], out_hbm.at[wid])


@functools.cache
def _sc_kernels():
    mesh = plsc.VectorSubcoreMesh(core_axis_name="c", subcore_axis_name="s")
    params = pltpu.CompilerParams(needs_layout_passes=False)
    hist_top = pl.kernel(
        _sc_hist_top_body,
        out_type=jax.ShapeDtypeStruct((NW, NBINS), jnp.int32),
        mesh=mesh,
        compiler_params=params,
        scratch_types=[
            pltpu.VMEM((CHUNK,), jnp.uint32),
            pltpu.VMEM((NBINS,), jnp.int32),
        ],
    )
    hist_low = pl.kernel(
        _sc_hist_low_body,
        out_type=jax.ShapeDtypeStruct((NW, NBINS), jnp.int32),
        mesh=mesh,
        compiler_params=params,
        scratch_types=[
            pltpu.VMEM((CHUNK,), jnp.uint32),
            pltpu.VMEM((NBINS + L,), jnp.int32),
            pltpu.VMEM((128,), jnp.int32),
        ],
    )
    return hist_top, hist_low


def _cumsum_flat(h):
    x = h
    for d in (1, 2, 4, 8, 16, 32, 64):
        x = x + jnp.concatenate(
            [jnp.zeros((512, d), jnp.int32), x[:, : 128 - d]], axis=1
        )
    rs = x[:, 127:128]
    ro = rs
    for d in (1, 2, 4, 8, 16, 32, 64, 128, 256):
        ro = ro + jnp.concatenate(
            [jnp.zeros((d, 1), jnp.int32), ro[: 512 - d, :]], axis=0
        )
    return x + (ro - rs)


def _sel_top_body(h_ref, bsel_ref, meta_ref):
    h = jnp.sum(h_ref[...], axis=0)
    cum = _cumsum_flat(h)
    row = lax.broadcasted_iota(jnp.int32, (512, 128), 0)
    col = lax.broadcasted_iota(jnp.int32, (512, 128), 1)
    flat = row * 128 + col
    n = jnp.sum(jnp.where(flat < 32768, h, 0))
    k = lax.div(n - 1, jnp.int32(2))
    le = cum <= k
    b = jnp.sum(le.astype(jnp.int32))
    excl = jnp.max(jnp.where(le, cum, 0))
    r = k - excl
    bsel_ref[...] = jnp.zeros((8, 128), jnp.int32) + b
    riota = lax.broadcasted_iota(jnp.int32, (8, 128), 0)
    meta_ref[...] = jnp.where(riota == 0, r, n)


def _select_top(h1):
    return pl.pallas_call(
        _sel_top_body,
        in_specs=[pl.BlockSpec((NW, 512, 128), lambda: (0, 0, 0))],
        out_specs=[
            pl.BlockSpec((8, 128), lambda: (0, 0)),
            pl.BlockSpec((8, 128), lambda: (0, 0)),
        ],
        out_shape=[
            jax.ShapeDtypeStruct((8, 128), jnp.int32),
            jax.ShapeDtypeStruct((8, 128), jnp.int32),
        ],
    )(h1)


def _sel_low_body(h_ref, bsel_ref, meta_ref, med_ref):
    h = jnp.sum(h_ref[...], axis=0)
    cum = _cumsum_flat(h)
    r = meta_ref[0, 0]
    n = meta_ref[1, 0]
    b = bsel_ref[0, 0]
    c = jnp.sum((cum <= r).astype(jnp.int32))
    med = jnp.where(n == 0, jnp.int32(-1), (b << 16) | c)
    med_ref[...] = jnp.zeros((8, 128), jnp.int32) + med


def _select_low(h2, bsel, meta):
    return pl.pallas_call(
        _sel_low_body,
        in_specs=[
            pl.BlockSpec((NW, 512, 128), lambda: (0, 0, 0)),
            pl.BlockSpec((8, 128), lambda: (0, 0)),
            pl.BlockSpec((8, 128), lambda: (0, 0)),
        ],
        out_specs=pl.BlockSpec((8, 128), lambda: (0, 0)),
        out_shape=jax.ShapeDtypeStruct((8, 128), jnp.int32),
    )(h2, bsel, meta)


def _eq_body(combo_ref, med_ref, out_ref):
    med = med_ref[0, 0]
    bits = lax.bitcast_convert_type(
        combo_ref[...] & jnp.uint32(0x7FFFFFFF), jnp.int32
    )
    out_ref[...] = (bits == med).astype(jnp.int32)


def _compare(combo, med):
    rows = 64
    grid = (B, H // rows)
    return pl.pallas_call(
        _eq_body,
        grid=grid,
        in_specs=[
            pl.BlockSpec((1, rows, W), lambda i, j: (i, j, 0)),
            pl.BlockSpec((8, 128), lambda i, j: (0, 0)),
        ],
        out_specs=pl.BlockSpec((1, rows, W), lambda i, j: (i, j, 0)),
        out_shape=jax.ShapeDtypeStruct((B, H, W), jnp.int32),
    )(combo, med)


def kernel(img):
    hist_top, hist_low = _sc_kernels()
    combo = _make_combo(img)
    combo_flat = combo.reshape(N)
    h1 = hist_top(combo_flat).reshape(NW, 512, 128)
    bsel, meta = _select_top(h1)
    h2 = hist_low(combo_flat, bsel).reshape(NW, 512, 128)
    med = _select_low(h2, bsel, meta)
    res = _compare(combo, med)
    return res.reshape(B, 1, H, W)


# speedup vs baseline: 11.0337x; 1.1220x over previous
import functools

import jax
import jax.numpy as jnp
from jax import lax
from jax.experimental import pallas as pl
from jax.experimental.pallas import tpu as pltpu
from jax.experimental.pallas import tpu_sc as plsc

B, C, H, W = 8, 3, 512, 512
N = B * H * W

NC, NS, L = 2, 16, 16
NW = NC * NS
PER_TILE = N // NW
CHUNK = 8192
NCHUNKS = PER_TILE // CHUNK
UNROLL = 8

HR, HC = 512, 128


def _combo_body(img_ref, out_ref):
    x = img_ref[0]
    r, g, b = x[0], x[1], x[2]
    gray = (r * jnp.float32(0.299) + g * jnp.float32(0.587)) + b * jnp.float32(0.114)
    mean = (r + g + b) / jnp.float32(3.0)
    bits = lax.bitcast_convert_type(gray, jnp.uint32)
    combo = jnp.where(mean > jnp.float32(0.5), bits, bits | jnp.uint32(0x80000000))
    out_ref[0] = combo


def _make_combo(img):
    rows = 64
    grid = (B, H // rows)
    return pl.pallas_call(
        _combo_body,
        grid=grid,
        in_specs=[pl.BlockSpec((1, C, rows, W), lambda i, j: (i, 0, j, 0))],
        out_specs=pl.BlockSpec((1, rows, W), lambda i, j: (i, j, 0)),
        out_shape=jax.ShapeDtypeStruct((B, H, W), jnp.uint32),
    )(img)


def _zero_2d(hist_v, nrows):
    z = jnp.zeros((L,), jnp.int32)

    def zb(r, _):
        for j in range(HC // L):
            hist_v[r, pl.ds(j * L, L)] = z
        return 0

    lax.fori_loop(0, nrows, zb, 0)


def _hist_scan(combo_hbm, base, stages, sems, bin_fn):
    def start(ci):
        return pltpu.async_copy(
            combo_hbm.at[pl.ds(base + ci * CHUNK, CHUNK)],
            stages[ci % 2],
            sems[ci % 2],
        )

    handles = [start(0), None]
    for ci in range(NCHUNKS):
        if ci + 1 < NCHUNKS:
            handles[(ci + 1) % 2] = start(ci + 1)
        handles[ci % 2].wait()
        st = stages[ci % 2]

        def vec_body(vi, _):
            for j in range(UNROLL):
                bin_fn(st[pl.ds((vi * UNROLL + j) * L, L)])
            return 0

        lax.fori_loop(0, CHUNK // (L * UNROLL), vec_body, 0)


def _sc_hist_top_body(combo_hbm, out_hbm, stage0, stage1, hist_v, sem0, sem1):
    wid = lax.axis_index("s") * NC + lax.axis_index("c")
    base = wid * PER_TILE

    _zero_2d(hist_v, HR)

    def bin_fn(k16):
        bins = lax.convert_element_type(k16 >> 16, jnp.int32)
        cnts, last = plsc.scan_count(bins)
        plsc.addupdate_scatter(hist_v, [bins >> 7, bins & 127], cnts, mask=last)

    _hist_scan(combo_hbm, base, (stage0, stage1), (sem0, sem1), bin_fn)
    pltpu.sync_copy(hist_v, out_hbm.at[wid])


def _sc_hist_low_body(
    combo_hbm, bsel_hbm, out_hbm, stage0, stage1, hist_v, brow_v, sem0, sem1
):
    wid = lax.axis_index("s") * NC + lax.axis_index("c")
    base = wid * PER_TILE

    _zero_2d(hist_v, HR + 8)
    pltpu.sync_copy(bsel_hbm.at[0], brow_v)
    bv = brow_v[pl.ds(0, L)]

    jrow = jnp.full((L,), HR, jnp.int32)
    jcol = jnp.zeros((L,), jnp.int32)

    def bin_fn(k16):
        top = lax.convert_element_type(k16 >> 16, jnp.int32)
        low = lax.convert_element_type(k16 & jnp.uint32(0xFFFF), jnp.int32)
        match = top == bv
        rowi = jnp.where(match, low >> 7, jrow)
        coli = jnp.where(match, low & 127, jcol)
        cnts, last = plsc.scan_count(jnp.where(match, low, jnp.int32(-1)))
        plsc.addupdate_scatter(hist_v, [rowi, coli], cnts, mask=last)

    _hist_scan(combo_hbm, base, (stage0, stage1), (sem0, sem1), bin_fn)
    pltpu.sync_copy(hist_v.at[pl.ds(0, HR)], out_hbm.at[wid])


@functools.cache
def _sc_kernels():
    mesh = plsc.VectorSubcoreMesh(core_axis_name="c", subcore_axis_name="s")
    params = pltpu.CompilerParams(needs_layout_passes=False)
    hist_top = pl.kernel(
        _sc_hist_top_body,
        out_type=jax.ShapeDtypeStruct((NW, HR, HC), jnp.int32),
        mesh=mesh,
        compiler_params=params,
        scratch_types=[
            pltpu.VMEM((CHUNK,), jnp.uint32),
            pltpu.VMEM((CHUNK,), jnp.uint32),
            pltpu.VMEM((HR, HC), jnp.int32),
            pltpu.SemaphoreType.DMA,
            pltpu.SemaphoreType.DMA,
        ],
    )
    hist_low = pl.kernel(
        _sc_hist_low_body,
        out_type=jax.ShapeDtypeStruct((NW, HR, HC), jnp.int32),
        mesh=mesh,
        compiler_params=params,
        scratch_types=[
            pltpu.VMEM((CHUNK,), jnp.uint32),
            pltpu.VMEM((CHUNK,), jnp.uint32),
            pltpu.VMEM((HR + 8, HC), jnp.int32),
            pltpu.VMEM((128,), jnp.int32),
            pltpu.SemaphoreType.DMA,
            pltpu.SemaphoreType.DMA,
        ],
    )
    return hist_top, hist_low


def _cumsum_flat(h):
    x = h
    for d in (1, 2, 4, 8, 16, 32, 64):
        x = x + jnp.concatenate(
            [jnp.zeros((HR, d), jnp.int32), x[:, : HC - d]], axis=1
        )
    rs = x[:, HC - 1 : HC]
    ro = rs
    for d in (1, 2, 4, 8, 16, 32, 64, 128, 256):
        ro = ro + jnp.concatenate(
            [jnp.zeros((d, 1), jnp.int32), ro[: HR - d, :]], axis=0
        )
    return x + (ro - rs)


def _sel_top_body(h_ref, bsel_ref, meta_ref):
    h = jnp.sum(h_ref[...], axis=0)
    cum = _cumsum_flat(h)
    row = lax.broadcasted_iota(jnp.int32, (HR, HC), 0)
    col = lax.broadcasted_iota(jnp.int32, (HR, HC), 1)
    flat = row * HC + col
    n = jnp.sum(jnp.where(flat < 32768, h, 0))
    k = lax.div(n - 1, jnp.int32(2))
    le = cum <= k
    b = jnp.sum(le.astype(jnp.int32))
    excl = jnp.max(jnp.where(le, cum, 0))
    r = k - excl
    bsel_ref[...] = jnp.zeros((8, 128), jnp.int32) + b
    riota = lax.broadcasted_iota(jnp.int32, (8, 128), 0)
    meta_ref[...] = jnp.where(riota == 0, r, n)


def _select_top(h1):
    return pl.pallas_call(
        _sel_top_body,
        in_specs=[pl.BlockSpec((NW, HR, HC), lambda: (0, 0, 0))],
        out_specs=[
            pl.BlockSpec((8, 128), lambda: (0, 0)),
            pl.BlockSpec((8, 128), lambda: (0, 0)),
        ],
        out_shape=[
            jax.ShapeDtypeStruct((8, 128), jnp.int32),
            jax.ShapeDtypeStruct((8, 128), jnp.int32),
        ],
    )(h1)


def _sel_low_body(h_ref, bsel_ref, meta_ref, med_ref):
    h = jnp.sum(h_ref[...], axis=0)
    cum = _cumsum_flat(h)
    r = meta_ref[0, 0]
    n = meta_ref[1, 0]
    b = bsel_ref[0, 0]
    c = jnp.sum((cum <= r).astype(jnp.int32))
    med = jnp.where(n == 0, jnp.int32(-1), (b << 16) | c)
    med_ref[...] = jnp.zeros((8, 128), jnp.int32) + med


def _select_low(h2, bsel, meta):
    return pl.pallas_call(
        _sel_low_body,
        in_specs=[
            pl.BlockSpec((NW, HR, HC), lambda: (0, 0, 0)),
            pl.BlockSpec((8, 128), lambda: (0, 0)),
            pl.BlockSpec((8, 128), lambda: (0, 0)),
        ],
        out_specs=pl.BlockSpec((8, 128), lambda: (0, 0)),
        out_shape=jax.ShapeDtypeStruct((8, 128), jnp.int32),
    )(h2, bsel, meta)


def _eq_body(combo_ref, med_ref, out_ref):
    med = med_ref[0, 0]
    bits = lax.bitcast_convert_type(
        combo_ref[...] & jnp.uint32(0x7FFFFFFF), jnp.int32
    )
    out_ref[...] = (bits == med).astype(jnp.int32)


def _compare(combo, med):
    rows = 64
    grid = (B, H // rows)
    return pl.pallas_call(
        _eq_body,
        grid=grid,
        in_specs=[
            pl.BlockSpec((1, rows, W), lambda i, j: (i, j, 0)),
            pl.BlockSpec((8, 128), lambda i, j: (0, 0)),
        ],
        out_specs=pl.BlockSpec((1, rows, W), lambda i, j: (i, j, 0)),
        out_shape=jax.ShapeDtypeStruct((B, H, W), jnp.int32),
    )(combo, med)


def kernel(img):
    hist_top, hist_low = _sc_kernels()
    combo = _make_combo(img)
    combo_flat = combo.reshape(N)
    h1 = hist_top(combo_flat)
    bsel, meta = _select_top(h1)
    h2 = hist_low(combo_flat, bsel)
    med = _select_low(h2, bsel, meta)
    res = _compare(combo, med)
    return res.reshape(B, 1, H, W)


# speedup vs baseline: 13.0699x; 1.1845x over previous
import functools

import jax
import jax.numpy as jnp
from jax import lax
from jax.experimental import pallas as pl
from jax.experimental.pallas import tpu as pltpu
from jax.experimental.pallas import tpu_sc as plsc

B, C, H, W = 8, 3, 512, 512
N = B * H * W

NC, NS, L = 2, 16, 16
NW = NC * NS
PER_TILE = N // NW
CHUNK = 8192
NCHUNKS = PER_TILE // CHUNK
UNROLL = 8

HR, HC = 512, 128


def _combo_body(img_ref, out_ref):
    x = img_ref[0]
    r, g, b = x[0], x[1], x[2]
    gray = (r * jnp.float32(0.299) + g * jnp.float32(0.587)) + b * jnp.float32(0.114)
    mean = (r + g + b) / jnp.float32(3.0)
    bits = lax.bitcast_convert_type(gray, jnp.uint32)
    combo = jnp.where(mean > jnp.float32(0.5), bits, bits | jnp.uint32(0x80000000))
    out_ref[0] = combo


def _make_combo(img):
    rows = 64
    grid = (B, H // rows)
    return pl.pallas_call(
        _combo_body,
        grid=grid,
        in_specs=[pl.BlockSpec((1, C, rows, W), lambda i, j: (i, 0, j, 0))],
        out_specs=pl.BlockSpec((1, rows, W), lambda i, j: (i, j, 0)),
        out_shape=jax.ShapeDtypeStruct((B, H, W), jnp.uint32),
    )(img)


def _zero_2d(hist_v, nrows):
    z = jnp.zeros((L,), jnp.int32)

    def zb(r, _):
        for j in range(HC // L):
            hist_v[r, pl.ds(j * L, L)] = z
        return 0

    lax.fori_loop(0, nrows, zb, 0)


def _hist_scan(combo_hbm, base, stages, sems, bin_fn):
    def start(ci):
        return pltpu.async_copy(
            combo_hbm.at[pl.ds(base + ci * CHUNK, CHUNK)],
            stages[ci % 2],
            sems[ci % 2],
        )

    handles = [start(0), None]
    for ci in range(NCHUNKS):
        if ci + 1 < NCHUNKS:
            handles[(ci + 1) % 2] = start(ci + 1)
        handles[ci % 2].wait()
        st = stages[ci % 2]

        def vec_body(vi, _):
            for j in range(UNROLL):
                bin_fn(st[pl.ds((vi * UNROLL + j) * L, L)])
            return 0

        lax.fori_loop(0, CHUNK // (L * UNROLL), vec_body, 0)


def _sc_hist_top_body(combo_hbm, out_hbm, stage0, stage1, hist_v, sem0, sem1):
    wid = lax.axis_index("s") * NC + lax.axis_index("c")
    base = wid * PER_TILE

    _zero_2d(hist_v, HR)

    ones = jnp.ones((L,), jnp.int32)

    def bin_fn(k16):
        bins = lax.convert_element_type(k16 >> 16, jnp.int32)
        plsc.addupdate_scatter(hist_v, [bins >> 7, bins & 127], ones)

    _hist_scan(combo_hbm, base, (stage0, stage1), (sem0, sem1), bin_fn)
    pltpu.sync_copy(hist_v, out_hbm.at[wid])


def _sc_hist_low_body(
    combo_hbm, bsel_hbm, out_hbm, stage0, stage1, hist_v, brow_v, sem0, sem1
):
    wid = lax.axis_index("s") * NC + lax.axis_index("c")
    base = wid * PER_TILE

    _zero_2d(hist_v, HR + 8)
    pltpu.sync_copy(bsel_hbm.at[0], brow_v)
    bv = brow_v[pl.ds(0, L)]

    jrow = jnp.full((L,), HR, jnp.int32)
    jcol = jnp.zeros((L,), jnp.int32)

    ones = jnp.ones((L,), jnp.int32)

    def bin_fn(k16):
        top = lax.convert_element_type(k16 >> 16, jnp.int32)
        low = lax.convert_element_type(k16 & jnp.uint32(0xFFFF), jnp.int32)
        match = top == bv
        rowi = jnp.where(match, low >> 7, jrow)
        coli = jnp.where(match, low & 127, jcol)
        plsc.addupdate_scatter(hist_v, [rowi, coli], ones)

    _hist_scan(combo_hbm, base, (stage0, stage1), (sem0, sem1), bin_fn)
    pltpu.sync_copy(hist_v.at[pl.ds(0, HR)], out_hbm.at[wid])


@functools.cache
def _sc_kernels():
    mesh = plsc.VectorSubcoreMesh(core_axis_name="c", subcore_axis_name="s")
    params = pltpu.CompilerParams(needs_layout_passes=False)
    hist_top = pl.kernel(
        _sc_hist_top_body,
        out_type=jax.ShapeDtypeStruct((NW, HR, HC), jnp.int32),
        mesh=mesh,
        compiler_params=params,
        scratch_types=[
            pltpu.VMEM((CHUNK,), jnp.uint32),
            pltpu.VMEM((CHUNK,), jnp.uint32),
            pltpu.VMEM((HR, HC), jnp.int32),
            pltpu.SemaphoreType.DMA,
            pltpu.SemaphoreType.DMA,
        ],
    )
    hist_low = pl.kernel(
        _sc_hist_low_body,
        out_type=jax.ShapeDtypeStruct((NW, HR, HC), jnp.int32),
        mesh=mesh,
        compiler_params=params,
        scratch_types=[
            pltpu.VMEM((CHUNK,), jnp.uint32),
            pltpu.VMEM((CHUNK,), jnp.uint32),
            pltpu.VMEM((HR + 8, HC), jnp.int32),
            pltpu.VMEM((128,), jnp.int32),
            pltpu.SemaphoreType.DMA,
            pltpu.SemaphoreType.DMA,
        ],
    )
    return hist_top, hist_low


def _cumsum_flat(h):
    x = h
    for d in (1, 2, 4, 8, 16, 32, 64):
        x = x + jnp.concatenate(
            [jnp.zeros((HR, d), jnp.int32), x[:, : HC - d]], axis=1
        )
    rs = x[:, HC - 1 : HC]
    ro = rs
    for d in (1, 2, 4, 8, 16, 32, 64, 128, 256):
        ro = ro + jnp.concatenate(
            [jnp.zeros((d, 1), jnp.int32), ro[: HR - d, :]], axis=0
        )
    return x + (ro - rs)


def _sel_top_body(h_ref, bsel_ref, meta_ref):
    h = jnp.sum(h_ref[...], axis=0)
    cum = _cumsum_flat(h)
    row = lax.broadcasted_iota(jnp.int32, (HR, HC), 0)
    col = lax.broadcasted_iota(jnp.int32, (HR, HC), 1)
    flat = row * HC + col
    n = jnp.sum(jnp.where(flat < 32768, h, 0))
    k = lax.div(n - 1, jnp.int32(2))
    le = cum <= k
    b = jnp.sum(le.astype(jnp.int32))
    excl = jnp.max(jnp.where(le, cum, 0))
    r = k - excl
    bsel_ref[...] = jnp.zeros((8, 128), jnp.int32) + b
    riota = lax.broadcasted_iota(jnp.int32, (8, 128), 0)
    meta_ref[...] = jnp.where(riota == 0, r, n)


def _select_top(h1):
    return pl.pallas_call(
        _sel_top_body,
        in_specs=[pl.BlockSpec((NW, HR, HC), lambda: (0, 0, 0))],
        out_specs=[
            pl.BlockSpec((8, 128), lambda: (0, 0)),
            pl.BlockSpec((8, 128), lambda: (0, 0)),
        ],
        out_shape=[
            jax.ShapeDtypeStruct((8, 128), jnp.int32),
            jax.ShapeDtypeStruct((8, 128), jnp.int32),
        ],
    )(h1)


def _sel_low_body(h_ref, bsel_ref, meta_ref, med_ref):
    h = jnp.sum(h_ref[...], axis=0)
    cum = _cumsum_flat(h)
    r = meta_ref[0, 0]
    n = meta_ref[1, 0]
    b = bsel_ref[0, 0]
    c = jnp.sum((cum <= r).astype(jnp.int32))
    med = jnp.where(n == 0, jnp.int32(-1), (b << 16) | c)
    med_ref[...] = jnp.zeros((8, 128), jnp.int32) + med


def _select_low(h2, bsel, meta):
    return pl.pallas_call(
        _sel_low_body,
        in_specs=[
            pl.BlockSpec((NW, HR, HC), lambda: (0, 0, 0)),
            pl.BlockSpec((8, 128), lambda: (0, 0)),
            pl.BlockSpec((8, 128), lambda: (0, 0)),
        ],
        out_specs=pl.BlockSpec((8, 128), lambda: (0, 0)),
        out_shape=jax.ShapeDtypeStruct((8, 128), jnp.int32),
    )(h2, bsel, meta)


def _eq_body(combo_ref, med_ref, out_ref):
    med = med_ref[0, 0]
    bits = lax.bitcast_convert_type(
        combo_ref[...] & jnp.uint32(0x7FFFFFFF), jnp.int32
    )
    out_ref[...] = (bits == med).astype(jnp.int32)


def _compare(combo, med):
    rows = 64
    grid = (B, H // rows)
    return pl.pallas_call(
        _eq_body,
        grid=grid,
        in_specs=[
            pl.BlockSpec((1, rows, W), lambda i, j: (i, j, 0)),
            pl.BlockSpec((8, 128), lambda i, j: (0, 0)),
        ],
        out_specs=pl.BlockSpec((1, rows, W), lambda i, j: (i, j, 0)),
        out_shape=jax.ShapeDtypeStruct((B, H, W), jnp.int32),
    )(combo, med)


def kernel(img):
    hist_top, hist_low = _sc_kernels()
    combo = _make_combo(img)
    combo_flat = combo.reshape(N)
    h1 = hist_top(combo_flat)
    bsel, meta = _select_top(h1)
    h2 = hist_low(combo_flat, bsel)
    med = _select_low(h2, bsel, meta)
    res = _compare(combo, med)
    return res.reshape(B, 1, H, W)


# speedup vs baseline: 13.3654x; 1.0226x over previous
import functools

import jax
import jax.numpy as jnp
from jax import lax
from jax.experimental import pallas as pl
from jax.experimental.pallas import tpu as pltpu
from jax.experimental.pallas import tpu_sc as plsc

B, C, H, W = 8, 3, 512, 512
N = B * H * W

NC, NS, L = 2, 16, 16
NW = NC * NS
PER_TILE = N // NW
CHUNK = 8192
NCHUNKS = PER_TILE // CHUNK
UNROLL = 8

HR, HC = 512, 128


def _combo_body(img_ref, out_ref):
    x = img_ref[0]
    r, g, b = x[0], x[1], x[2]
    gray = (r * jnp.float32(0.299) + g * jnp.float32(0.587)) + b * jnp.float32(0.114)
    mean = (r + g + b) / jnp.float32(3.0)
    bits = lax.bitcast_convert_type(gray, jnp.uint32)
    combo = jnp.where(mean > jnp.float32(0.5), bits, bits | jnp.uint32(0x80000000))
    out_ref[0] = combo


def _make_combo(img):
    rows = 64
    grid = (B, H // rows)
    return pl.pallas_call(
        _combo_body,
        grid=grid,
        in_specs=[pl.BlockSpec((1, C, rows, W), lambda i, j: (i, 0, j, 0))],
        out_specs=pl.BlockSpec((1, rows, W), lambda i, j: (i, j, 0)),
        out_shape=jax.ShapeDtypeStruct((B, H, W), jnp.uint32),
    )(img)


def _zero_2d(hist_v, nrows):
    z = jnp.zeros((L,), jnp.int32)

    def zb(r, _):
        for j in range(HC // L):
            hist_v[r, pl.ds(j * L, L)] = z
        return 0

    lax.fori_loop(0, nrows, zb, 0)


def _hist_scan(combo_hbm, base, stages, sems, bin_fn):
    def start(ci):
        return pltpu.async_copy(
            combo_hbm.at[pl.ds(base + ci * CHUNK, CHUNK)],
            stages[ci % 2],
            sems[ci % 2],
        )

    handles = [start(0), None]
    for ci in range(NCHUNKS):
        if ci + 1 < NCHUNKS:
            handles[(ci + 1) % 2] = start(ci + 1)
        handles[ci % 2].wait()
        st = stages[ci % 2]

        def vec_body(vi, _):
            for j in range(UNROLL):
                bin_fn(st[pl.ds((vi * UNROLL + j) * L, L)])
            return 0

        lax.fori_loop(0, CHUNK // (L * UNROLL), vec_body, 0)


def _sc_hist_top_body(combo_hbm, out_hbm, stage0, stage1, hist_v, sem0, sem1):
    wid = lax.axis_index("s") * NC + lax.axis_index("c")
    base = wid * PER_TILE

    _zero_2d(hist_v, HR)

    ones = jnp.ones((L,), jnp.int32)

    def bin_fn(k16):
        bins = lax.convert_element_type(k16 >> 16, jnp.int32)
        plsc.addupdate_scatter(hist_v, [bins >> 7, bins & 127], ones)

    _hist_scan(combo_hbm, base, (stage0, stage1), (sem0, sem1), bin_fn)
    pltpu.sync_copy(hist_v, out_hbm.at[wid])


def _sc_hist_low_body(
    combo_hbm, bsel_hbm, out_hbm, stage0, stage1, hist_v, brow_v, sem0, sem1
):
    wid = lax.axis_index("s") * NC + lax.axis_index("c")
    base = wid * PER_TILE

    _zero_2d(hist_v, HR + 8)
    pltpu.sync_copy(bsel_hbm.at[0], brow_v)
    bv = brow_v[pl.ds(0, L)]
    btop = lax.convert_element_type(bv, jnp.uint32) << 16
    junk = jnp.full((L,), 65536, jnp.uint32)
    ones = jnp.ones((L,), jnp.int32)

    def bin_fn(k16):
        flat = lax.convert_element_type(jnp.minimum(k16 ^ btop, junk), jnp.int32)
        plsc.addupdate_scatter(hist_v, [flat >> 7, flat & 127], ones)

    _hist_scan(combo_hbm, base, (stage0, stage1), (sem0, sem1), bin_fn)
    pltpu.sync_copy(hist_v.at[pl.ds(0, HR)], out_hbm.at[wid])


@functools.cache
def _sc_kernels():
    mesh = plsc.VectorSubcoreMesh(core_axis_name="c", subcore_axis_name="s")
    params = pltpu.CompilerParams(needs_layout_passes=False)
    hist_top = pl.kernel(
        _sc_hist_top_body,
        out_type=jax.ShapeDtypeStruct((NW, HR, HC), jnp.int32),
        mesh=mesh,
        compiler_params=params,
        scratch_types=[
            pltpu.VMEM((CHUNK,), jnp.uint32),
            pltpu.VMEM((CHUNK,), jnp.uint32),
            pltpu.VMEM((HR, HC), jnp.int32),
            pltpu.SemaphoreType.DMA,
            pltpu.SemaphoreType.DMA,
        ],
    )
    hist_low = pl.kernel(
        _sc_hist_low_body,
        out_type=jax.ShapeDtypeStruct((NW, HR, HC), jnp.int32),
        mesh=mesh,
        compiler_params=params,
        scratch_types=[
            pltpu.VMEM((CHUNK,), jnp.uint32),
            pltpu.VMEM((CHUNK,), jnp.uint32),
            pltpu.VMEM((HR + 8, HC), jnp.int32),
            pltpu.VMEM((128,), jnp.int32),
            pltpu.SemaphoreType.DMA,
            pltpu.SemaphoreType.DMA,
        ],
    )
    return hist_top, hist_low


def _cumsum_flat(h):
    x = h
    for d in (1, 2, 4, 8, 16, 32, 64):
        x = x + jnp.concatenate(
            [jnp.zeros((HR, d), jnp.int32), x[:, : HC - d]], axis=1
        )
    rs = x[:, HC - 1 : HC]
    ro = rs
    for d in (1, 2, 4, 8, 16, 32, 64, 128, 256):
        ro = ro + jnp.concatenate(
            [jnp.zeros((d, 1), jnp.int32), ro[: HR - d, :]], axis=0
        )
    return x + (ro - rs)


def _sel_top_body(h_ref, bsel_ref, meta_ref):
    h = jnp.sum(h_ref[...], axis=0)
    cum = _cumsum_flat(h)
    row = lax.broadcasted_iota(jnp.int32, (HR, HC), 0)
    col = lax.broadcasted_iota(jnp.int32, (HR, HC), 1)
    flat = row * HC + col
    n = jnp.sum(jnp.where(flat < 32768, h, 0))
    k = lax.div(n - 1, jnp.int32(2))
    le = cum <= k
    b = jnp.sum(le.astype(jnp.int32))
    excl = jnp.max(jnp.where(le, cum, 0))
    r = k - excl
    bsel_ref[...] = jnp.zeros((8, 128), jnp.int32) + b
    riota = lax.broadcasted_iota(jnp.int32, (8, 128), 0)
    meta_ref[...] = jnp.where(riota == 0, r, n)


def _select_top(h1):
    return pl.pallas_call(
        _sel_top_body,
        in_specs=[pl.BlockSpec((NW, HR, HC), lambda: (0, 0, 0))],
        out_specs=[
            pl.BlockSpec((8, 128), lambda: (0, 0)),
            pl.BlockSpec((8, 128), lambda: (0, 0)),
        ],
        out_shape=[
            jax.ShapeDtypeStruct((8, 128), jnp.int32),
            jax.ShapeDtypeStruct((8, 128), jnp.int32),
        ],
    )(h1)


def _eq_body(h2_ref, bsel_ref, meta_ref, combo_ref, out_ref, med_sm):
    i = pl.program_id(0)
    j = pl.program_id(1)

    @pl.when((i == 0) & (j == 0))
    def _():
        h = jnp.sum(h2_ref[...], axis=0)
        cum = _cumsum_flat(h)
        r = meta_ref[0, 0]
        n = meta_ref[1, 0]
        b = bsel_ref[0, 0]
        c = jnp.sum((cum <= r).astype(jnp.int32))
        med_sm[0] = jnp.where(n == 0, jnp.int32(-1), (b << 16) | c)

    med = med_sm[0]
    bits = lax.bitcast_convert_type(
        combo_ref[...] & jnp.uint32(0x7FFFFFFF), jnp.int32
    )
    out_ref[...] = (bits == med).astype(jnp.int32)


def _compare(h2, bsel, meta, combo):
    rows = 64
    grid = (B, H // rows)
    return pl.pallas_call(
        _eq_body,
        grid=grid,
        in_specs=[
            pl.BlockSpec((NW, HR, HC), lambda i, j: (0, 0, 0)),
            pl.BlockSpec((8, 128), lambda i, j: (0, 0)),
            pl.BlockSpec((8, 128), lambda i, j: (0, 0)),
            pl.BlockSpec((1, rows, W), lambda i, j: (i, j, 0)),
        ],
        out_specs=pl.BlockSpec((1, rows, W), lambda i, j: (i, j, 0)),
        out_shape=jax.ShapeDtypeStruct((B, H, W), jnp.int32),
        scratch_shapes=[pltpu.SMEM((1,), jnp.int32)],
    )(h2, bsel, meta, combo)


def kernel(img):
    hist_top, hist_low = _sc_kernels()
    combo = _make_combo(img)
    combo_flat = combo.reshape(N)
    h1 = hist_top(combo_flat)
    bsel, meta = _select_top(h1)
    h2 = hist_low(combo_flat, bsel)
    res = _compare(h2, bsel, meta, combo)
    return res.reshape(B, 1, H, W)


# speedup vs baseline: 20.4600x; 1.5308x over previous
import functools

import jax
import jax.numpy as jnp
from jax import lax
from jax.experimental import pallas as pl
from jax.experimental.pallas import tpu as pltpu
from jax.experimental.pallas import tpu_sc as plsc

B, C, H, W = 8, 3, 512, 512
N = B * H * W

NC, NS, L = 2, 16, 16
NW = NC * NS
PER_TILE = N // NW
CHUNK = 8192
NCHUNKS = PER_TILE // CHUNK
UNROLL = 8

HR, HC = 512, 128


def _combo_body(img_ref, out_ref):
    x = img_ref[0]
    r, g, b = x[0], x[1], x[2]
    gray = (r * jnp.float32(0.299) + g * jnp.float32(0.587)) + b * jnp.float32(0.114)
    mean = (r + g + b) / jnp.float32(3.0)
    bits = lax.bitcast_convert_type(gray, jnp.uint32)
    combo = jnp.where(mean > jnp.float32(0.5), bits, bits | jnp.uint32(0x80000000))
    out_ref[0] = combo


def _make_combo(img):
    rows = 64
    grid = (B, H // rows)
    return pl.pallas_call(
        _combo_body,
        grid=grid,
        in_specs=[pl.BlockSpec((1, C, rows, W), lambda i, j: (i, 0, j, 0))],
        out_specs=pl.BlockSpec((1, rows, W), lambda i, j: (i, j, 0)),
        out_shape=jax.ShapeDtypeStruct((B, H, W), jnp.uint32),
    )(img)


def _zero_2d(hist_v, nrows):
    z = jnp.zeros((L,), jnp.int32)

    def zb(r, _):
        for j in range(HC // L):
            hist_v[r, pl.ds(j * L, L)] = z
        return 0

    lax.fori_loop(0, nrows, zb, 0)


def _hist_scan(combo_hbm, base, stages, sems, bin_fn):
    def start(ci):
        return pltpu.async_copy(
            combo_hbm.at[pl.ds(base + ci * CHUNK, CHUNK)],
            stages[ci % 2],
            sems[ci % 2],
        )

    handles = [start(0), None]
    for ci in range(NCHUNKS):
        if ci + 1 < NCHUNKS:
            handles[(ci + 1) % 2] = start(ci + 1)
        handles[ci % 2].wait()
        st = stages[ci % 2]

        @plsc.parallel_loop(0, CHUNK // L, step=1, unroll=UNROLL)
        def _(vi):
            bin_fn(st[pl.ds(vi * L, L)])


def _sc_hist_top_body(combo_hbm, out_hbm, stage0, stage1, hist_v, sem0, sem1):
    wid = lax.axis_index("s") * NC + lax.axis_index("c")
    base = wid * PER_TILE

    _zero_2d(hist_v, HR)

    ones = jnp.ones((L,), jnp.int32)

    def bin_fn(k16):
        bins = lax.convert_element_type(k16 >> 16, jnp.int32)
        plsc.addupdate_scatter(hist_v, [bins >> 7, bins & 127], ones)

    _hist_scan(combo_hbm, base, (stage0, stage1), (sem0, sem1), bin_fn)
    pltpu.sync_copy(hist_v, out_hbm.at[wid])


def _sc_hist_low_body(
    combo_hbm, bsel_hbm, out_hbm, stage0, stage1, hist_v, brow_v, sem0, sem1
):
    wid = lax.axis_index("s") * NC + lax.axis_index("c")
    base = wid * PER_TILE

    _zero_2d(hist_v, HR + 8)
    pltpu.sync_copy(bsel_hbm.at[0], brow_v)
    bv = brow_v[pl.ds(0, L)]
    btop = lax.convert_element_type(bv, jnp.uint32) << 16
    junk = jnp.full((L,), 65536, jnp.uint32)
    ones = jnp.ones((L,), jnp.int32)

    def bin_fn(k16):
        diff = k16 ^ btop
        flat = lax.convert_element_type(jnp.minimum(diff, junk), jnp.int32)
        plsc.addupdate_scatter(
            hist_v, [flat >> 7, flat & 127], ones, mask=diff < junk
        )

    _hist_scan(combo_hbm, base, (stage0, stage1), (sem0, sem1), bin_fn)
    pltpu.sync_copy(hist_v.at[pl.ds(0, HR)], out_hbm.at[wid])


@functools.cache
def _sc_kernels():
    mesh = plsc.VectorSubcoreMesh(core_axis_name="c", subcore_axis_name="s")
    params = pltpu.CompilerParams(needs_layout_passes=False)
    hist_top = pl.kernel(
        _sc_hist_top_body,
        out_type=jax.ShapeDtypeStruct((NW, HR, HC), jnp.int32),
        mesh=mesh,
        compiler_params=params,
        scratch_types=[
            pltpu.VMEM((CHUNK,), jnp.uint32),
            pltpu.VMEM((CHUNK,), jnp.uint32),
            pltpu.VMEM((HR, HC), jnp.int32),
            pltpu.SemaphoreType.DMA,
            pltpu.SemaphoreType.DMA,
        ],
    )
    hist_low = pl.kernel(
        _sc_hist_low_body,
        out_type=jax.ShapeDtypeStruct((NW, HR, HC), jnp.int32),
        mesh=mesh,
        compiler_params=params,
        scratch_types=[
            pltpu.VMEM((CHUNK,), jnp.uint32),
            pltpu.VMEM((CHUNK,), jnp.uint32),
            pltpu.VMEM((HR + 8, HC), jnp.int32),
            pltpu.VMEM((128,), jnp.int32),
            pltpu.SemaphoreType.DMA,
            pltpu.SemaphoreType.DMA,
        ],
    )
    return hist_top, hist_low


def _cumsum_flat(h):
    x = h
    for d in (1, 2, 4, 8, 16, 32, 64):
        x = x + jnp.concatenate(
            [jnp.zeros((HR, d), jnp.int32), x[:, : HC - d]], axis=1
        )
    rs = x[:, HC - 1 : HC]
    ro = rs
    for d in (1, 2, 4, 8, 16, 32, 64, 128, 256):
        ro = ro + jnp.concatenate(
            [jnp.zeros((d, 1), jnp.int32), ro[: HR - d, :]], axis=0
        )
    return x + (ro - rs)


def _sel_top_body(h_ref, bsel_ref, meta_ref):
    h = jnp.sum(h_ref[...], axis=0)
    cum = _cumsum_flat(h)
    row = lax.broadcasted_iota(jnp.int32, (HR, HC), 0)
    col = lax.broadcasted_iota(jnp.int32, (HR, HC), 1)
    flat = row * HC + col
    n = jnp.sum(jnp.where(flat < 32768, h, 0))
    k = lax.div(n - 1, jnp.int32(2))
    le = cum <= k
    b = jnp.sum(le.astype(jnp.int32))
    excl = jnp.max(jnp.where(le, cum, 0))
    r = k - excl
    bsel_ref[...] = jnp.zeros((8, 128), jnp.int32) + b
    riota = lax.broadcasted_iota(jnp.int32, (8, 128), 0)
    meta_ref[...] = jnp.where(riota == 0, r, n)


def _select_top(h1):
    return pl.pallas_call(
        _sel_top_body,
        in_specs=[pl.BlockSpec((NW, HR, HC), lambda: (0, 0, 0))],
        out_specs=[
            pl.BlockSpec((8, 128), lambda: (0, 0)),
            pl.BlockSpec((8, 128), lambda: (0, 0)),
        ],
        out_shape=[
            jax.ShapeDtypeStruct((8, 128), jnp.int32),
            jax.ShapeDtypeStruct((8, 128), jnp.int32),
        ],
    )(h1)


def _eq_body(h2_ref, bsel_ref, meta_ref, combo_ref, out_ref, med_sm):
    i = pl.program_id(0)
    j = pl.program_id(1)

    @pl.when((i == 0) & (j == 0))
    def _():
        h = jnp.sum(h2_ref[...], axis=0)
        cum = _cumsum_flat(h)
        r = meta_ref[0, 0]
        n = meta_ref[1, 0]
        b = bsel_ref[0, 0]
        c = jnp.sum((cum <= r).astype(jnp.int32))
        med_sm[0] = jnp.where(n == 0, jnp.int32(-1), (b << 16) | c)

    med = med_sm[0]
    bits = lax.bitcast_convert_type(
        combo_ref[...] & jnp.uint32(0x7FFFFFFF), jnp.int32
    )
    out_ref[...] = (bits == med).astype(jnp.int32)


def _compare(h2, bsel, meta, combo):
    rows = 64
    grid = (B, H // rows)
    return pl.pallas_call(
        _eq_body,
        grid=grid,
        in_specs=[
            pl.BlockSpec((NW, HR, HC), lambda i, j: (0, 0, 0)),
            pl.BlockSpec((8, 128), lambda i, j: (0, 0)),
            pl.BlockSpec((8, 128), lambda i, j: (0, 0)),
            pl.BlockSpec((1, rows, W), lambda i, j: (i, j, 0)),
        ],
        out_specs=pl.BlockSpec((1, rows, W), lambda i, j: (i, j, 0)),
        out_shape=jax.ShapeDtypeStruct((B, H, W), jnp.int32),
        scratch_shapes=[pltpu.SMEM((1,), jnp.int32)],
    )(h2, bsel, meta, combo)


def kernel(img):
    hist_top, hist_low = _sc_kernels()
    combo = _make_combo(img)
    combo_flat = combo.reshape(N)
    h1 = hist_top(combo_flat)
    bsel, meta = _select_top(h1)
    h2 = hist_low(combo_flat, bsel)
    res = _compare(h2, bsel, meta, combo)
    return res.reshape(B, 1, H, W)


# speedup vs baseline: 29.9776x; 1.4652x over previous
import functools

import jax
import jax.numpy as jnp
from jax import lax
from jax.experimental import pallas as pl
from jax.experimental.pallas import tpu as pltpu
from jax.experimental.pallas import tpu_sc as plsc

B, C, H, W = 8, 3, 512, 512
N = B * H * W

NC, NS, L = 2, 16, 16
NW = NC * NS
PER_TILE = N // NW
CHUNK = 8192
NCHUNKS = PER_TILE // CHUNK
UNROLL = 8

HR, HC = 512, 128


def _combo_body(img_ref, out_ref):
    x = img_ref[0]
    r, g, b = x[0], x[1], x[2]
    gray = (r * jnp.float32(0.299) + g * jnp.float32(0.587)) + b * jnp.float32(0.114)
    bits = lax.bitcast_convert_type(gray, jnp.uint32)
    combo = jnp.where(
        (r + g + b) > jnp.float32(1.5), bits, bits | jnp.uint32(0x80000000)
    )
    out_ref[0] = combo


def _make_combo(img):
    rows = 256
    grid = (B, H // rows)
    return pl.pallas_call(
        _combo_body,
        grid=grid,
        in_specs=[pl.BlockSpec((1, C, rows, W), lambda i, j: (i, 0, j, 0))],
        out_specs=pl.BlockSpec((1, rows, W), lambda i, j: (i, j, 0)),
        out_shape=jax.ShapeDtypeStruct((B, H, W), jnp.uint32),
    )(img)


def _zero_2d(hist_v, nrows):
    z = jnp.zeros((L,), jnp.int32)

    def zb(r, _):
        for j in range(HC // L):
            hist_v[r, pl.ds(j * L, L)] = z
        return 0

    lax.fori_loop(0, nrows, zb, 0)


def _hist_scan(combo_hbm, base, stages, sems, bin_fn):
    def start(ci):
        return pltpu.async_copy(
            combo_hbm.at[pl.ds(base + ci * CHUNK, CHUNK)],
            stages[ci % 2],
            sems[ci % 2],
        )

    handles = [start(0), None]
    for ci in range(NCHUNKS):
        if ci + 1 < NCHUNKS:
            handles[(ci + 1) % 2] = start(ci + 1)
        handles[ci % 2].wait()
        st = stages[ci % 2]

        @plsc.parallel_loop(0, CHUNK // L, step=1, unroll=UNROLL)
        def _(vi):
            bin_fn(st[pl.ds(vi * L, L)])


def _sc_hist_top_body(combo_hbm, out_hbm, stage0, stage1, hist_v, sem0, sem1):
    wid = lax.axis_index("s") * NC + lax.axis_index("c")
    base = wid * PER_TILE

    _zero_2d(hist_v, HR)

    ones = jnp.ones((L,), jnp.int32)

    def bin_fn(k16):
        bins = lax.convert_element_type(k16 >> 16, jnp.int32)
        plsc.addupdate_scatter(hist_v, [bins >> 7, bins & 127], ones)

    _hist_scan(combo_hbm, base, (stage0, stage1), (sem0, sem1), bin_fn)
    pltpu.sync_copy(hist_v, out_hbm.at[wid])


def _sc_hist_low_body(
    combo_hbm, bsel_hbm, out_hbm, stage0, stage1, hist_v, brow_v, sem0, sem1
):
    wid = lax.axis_index("s") * NC + lax.axis_index("c")
    base = wid * PER_TILE

    _zero_2d(hist_v, HR + 8)
    pltpu.sync_copy(bsel_hbm.at[0], brow_v)
    bv = brow_v[pl.ds(0, L)]
    btop = lax.convert_element_type(bv, jnp.uint32) << 16
    junk = jnp.full((L,), 65536, jnp.uint32)
    ones = jnp.ones((L,), jnp.int32)

    def bin_fn(k16):
        diff = k16 ^ btop
        flat = lax.convert_element_type(jnp.minimum(diff, junk), jnp.int32)
        plsc.addupdate_scatter(
            hist_v, [flat >> 7, flat & 127], ones, mask=diff < junk
        )

    _hist_scan(combo_hbm, base, (stage0, stage1), (sem0, sem1), bin_fn)
    pltpu.sync_copy(hist_v.at[pl.ds(0, HR)], out_hbm.at[wid])


@functools.cache
def _sc_kernels():
    mesh = plsc.VectorSubcoreMesh(core_axis_name="c", subcore_axis_name="s")
    params = pltpu.CompilerParams(needs_layout_passes=False)
    hist_top = pl.kernel(
        _sc_hist_top_body,
        out_type=jax.ShapeDtypeStruct((NW, HR, HC), jnp.int32),
        mesh=mesh,
        compiler_params=params,
        scratch_types=[
            pltpu.VMEM((CHUNK,), jnp.uint32),
            pltpu.VMEM((CHUNK,), jnp.uint32),
            pltpu.VMEM((HR, HC), jnp.int32),
            pltpu.SemaphoreType.DMA,
            pltpu.SemaphoreType.DMA,
        ],
    )
    hist_low = pl.kernel(
        _sc_hist_low_body,
        out_type=jax.ShapeDtypeStruct((NW, HR, HC), jnp.int32),
        mesh=mesh,
        compiler_params=params,
        scratch_types=[
            pltpu.VMEM((CHUNK,), jnp.uint32),
            pltpu.VMEM((CHUNK,), jnp.uint32),
            pltpu.VMEM((HR + 8, HC), jnp.int32),
            pltpu.VMEM((128,), jnp.int32),
            pltpu.SemaphoreType.DMA,
            pltpu.SemaphoreType.DMA,
        ],
    )
    return hist_top, hist_low


def _cumsum_flat(h):
    x = h
    for d in (1, 2, 4, 8, 16, 32, 64):
        x = x + jnp.concatenate(
            [jnp.zeros((HR, d), jnp.int32), x[:, : HC - d]], axis=1
        )
    rs = x[:, HC - 1 : HC]
    ro = rs
    for d in (1, 2, 4, 8, 16, 32, 64, 128, 256):
        ro = ro + jnp.concatenate(
            [jnp.zeros((d, 1), jnp.int32), ro[: HR - d, :]], axis=0
        )
    return x + (ro - rs)


def _sel_top_body(h_ref, bsel_ref, meta_ref):
    h = jnp.sum(h_ref[...], axis=0)
    cum = _cumsum_flat(h)
    row = lax.broadcasted_iota(jnp.int32, (HR, HC), 0)
    col = lax.broadcasted_iota(jnp.int32, (HR, HC), 1)
    flat = row * HC + col
    n = jnp.sum(jnp.where(flat < 32768, h, 0))
    k = lax.div(n - 1, jnp.int32(2))
    le = cum <= k
    b = jnp.sum(le.astype(jnp.int32))
    excl = jnp.max(jnp.where(le, cum, 0))
    r = k - excl
    bsel_ref[...] = jnp.zeros((8, 128), jnp.int32) + b
    riota = lax.broadcasted_iota(jnp.int32, (8, 128), 0)
    meta_ref[...] = jnp.where(riota == 0, r, n)


def _select_top(h1):
    return pl.pallas_call(
        _sel_top_body,
        in_specs=[pl.BlockSpec((NW, HR, HC), lambda: (0, 0, 0))],
        out_specs=[
            pl.BlockSpec((8, 128), lambda: (0, 0)),
            pl.BlockSpec((8, 128), lambda: (0, 0)),
        ],
        out_shape=[
            jax.ShapeDtypeStruct((8, 128), jnp.int32),
            jax.ShapeDtypeStruct((8, 128), jnp.int32),
        ],
    )(h1)


def _eq_body(h2_ref, bsel_ref, meta_ref, combo_ref, out_ref, med_sm):
    i = pl.program_id(0)
    j = pl.program_id(1)

    @pl.when((i == 0) & (j == 0))
    def _():
        h = jnp.sum(h2_ref[...], axis=0)
        cum = _cumsum_flat(h)
        r = meta_ref[0, 0]
        n = meta_ref[1, 0]
        b = bsel_ref[0, 0]
        c = jnp.sum((cum <= r).astype(jnp.int32))
        med_sm[0] = jnp.where(n == 0, jnp.int32(-1), (b << 16) | c)

    med = med_sm[0]
    bits = lax.bitcast_convert_type(
        combo_ref[...] & jnp.uint32(0x7FFFFFFF), jnp.int32
    )
    out_ref[...] = (bits == med).astype(jnp.int32)


def _compare(h2, bsel, meta, combo):
    rows = 256
    grid = (B, H // rows)
    return pl.pallas_call(
        _eq_body,
        grid=grid,
        in_specs=[
            pl.BlockSpec((NW, HR, HC), lambda i, j: (0, 0, 0)),
            pl.BlockSpec((8, 128), lambda i, j: (0, 0)),
            pl.BlockSpec((8, 128), lambda i, j: (0, 0)),
            pl.BlockSpec((1, rows, W), lambda i, j: (i, j, 0)),
        ],
        out_specs=pl.BlockSpec((1, rows, W), lambda i, j: (i, j, 0)),
        out_shape=jax.ShapeDtypeStruct((B, H, W), jnp.int32),
        scratch_shapes=[pltpu.SMEM((1,), jnp.int32)],
    )(h2, bsel, meta, combo)


def kernel(img):
    hist_top, hist_low = _sc_kernels()
    combo = _make_combo(img)
    combo_flat = combo.reshape(N)
    h1 = hist_top(combo_flat)
    bsel, meta = _select_top(h1)
    h2 = hist_low(combo_flat, bsel)
    res = _compare(h2, bsel, meta, combo)
    return res.reshape(B, 1, H, W)


# speedup vs baseline: 33.7152x; 1.1247x over previous
import functools

import jax
import jax.numpy as jnp
from jax import lax
from jax.experimental import pallas as pl
from jax.experimental.pallas import tpu as pltpu
from jax.experimental.pallas import tpu_sc as plsc

B, C, H, W = 8, 3, 512, 512
N = B * H * W

NC, NS, L = 2, 16, 16
NW = NC * NS
PER_TILE = N // NW
CHUNK = 8192
NCHUNKS = PER_TILE // CHUNK
UNROLL = 8

HR, HC = 512, 128


def _combo_body(img_ref, out_ref):
    x = img_ref[0]
    r, g, b = x[0], x[1], x[2]
    gray = (r * jnp.float32(0.299) + g * jnp.float32(0.587)) + b * jnp.float32(0.114)
    bits = lax.bitcast_convert_type(gray, jnp.uint32)
    combo = jnp.where(
        (r + g + b) > jnp.float32(1.5), bits, bits | jnp.uint32(0x80000000)
    )
    out_ref[...] = combo.reshape(out_ref.shape)


def _make_combo(img):
    rows = 256
    grid = (B, H // rows)
    return pl.pallas_call(
        _combo_body,
        grid=grid,
        in_specs=[pl.BlockSpec((1, C, rows, W), lambda i, j: (i, 0, j, 0))],
        out_specs=pl.BlockSpec((rows * 4, 128), lambda i, j: (i * 2 + j, 0)),
        out_shape=jax.ShapeDtypeStruct((N // 128, 128), jnp.uint32),
    )(img)


def _zero_2d(hist_v, nrows):
    z = jnp.zeros((L,), jnp.int32)

    def zb(r, _):
        for j in range(HC // L):
            hist_v[r, pl.ds(j * L, L)] = z
        return 0

    lax.fori_loop(0, nrows, zb, 0)


def _hist_scan(combo_hbm, base, stages, sems, bin_fn):
    def start(ci):
        return pltpu.async_copy(
            combo_hbm.at[pl.ds(base + ci * CHUNK, CHUNK)],
            stages[ci % 2],
            sems[ci % 2],
        )

    handles = [start(0), None]
    for ci in range(NCHUNKS):
        if ci + 1 < NCHUNKS:
            handles[(ci + 1) % 2] = start(ci + 1)
        handles[ci % 2].wait()
        st = stages[ci % 2]

        @plsc.parallel_loop(0, CHUNK // L, step=1, unroll=UNROLL)
        def _(vi):
            bin_fn(st[pl.ds(vi * L, L)])


def _sc_hist_top_body(combo_hbm, out_hbm, stage0, stage1, hist_v, sem0, sem1):
    wid = lax.axis_index("s") * NC + lax.axis_index("c")
    base = wid * PER_TILE

    _zero_2d(hist_v, HR)

    ones = jnp.ones((L,), jnp.int32)

    def bin_fn(k16):
        bins = lax.convert_element_type(k16 >> 16, jnp.int32)
        plsc.addupdate_scatter(hist_v, [bins >> 7, bins & 127], ones)

    _hist_scan(combo_hbm, base, (stage0, stage1), (sem0, sem1), bin_fn)
    pltpu.sync_copy(hist_v, out_hbm.at[wid])


def _sc_hist_low_body(
    combo_hbm, bsel_hbm, out_hbm, stage0, stage1, hist_v, brow_v, sem0, sem1
):
    wid = lax.axis_index("s") * NC + lax.axis_index("c")
    base = wid * PER_TILE

    _zero_2d(hist_v, HR + 8)
    pltpu.sync_copy(bsel_hbm.at[0], brow_v)
    bv = brow_v[pl.ds(0, L)]
    btop = lax.convert_element_type(bv, jnp.uint32) << 16
    junk = jnp.full((L,), 65536, jnp.uint32)
    ones = jnp.ones((L,), jnp.int32)

    def bin_fn(k16):
        diff = k16 ^ btop
        flat = lax.convert_element_type(jnp.minimum(diff, junk), jnp.int32)
        plsc.addupdate_scatter(
            hist_v, [flat >> 7, flat & 127], ones, mask=diff < junk
        )

    _hist_scan(combo_hbm, base, (stage0, stage1), (sem0, sem1), bin_fn)
    pltpu.sync_copy(hist_v.at[pl.ds(0, HR)], out_hbm.at[wid])


@functools.cache
def _sc_kernels():
    mesh = plsc.VectorSubcoreMesh(core_axis_name="c", subcore_axis_name="s")
    params = pltpu.CompilerParams(needs_layout_passes=False)
    hist_top = pl.kernel(
        _sc_hist_top_body,
        out_type=jax.ShapeDtypeStruct((NW, HR, HC), jnp.int32),
        mesh=mesh,
        compiler_params=params,
        scratch_types=[
            pltpu.VMEM((CHUNK,), jnp.uint32),
            pltpu.VMEM((CHUNK,), jnp.uint32),
            pltpu.VMEM((HR, HC), jnp.int32),
            pltpu.SemaphoreType.DMA,
            pltpu.SemaphoreType.DMA,
        ],
    )
    hist_low = pl.kernel(
        _sc_hist_low_body,
        out_type=jax.ShapeDtypeStruct((NW, HR, HC), jnp.int32),
        mesh=mesh,
        compiler_params=params,
        scratch_types=[
            pltpu.VMEM((CHUNK,), jnp.uint32),
            pltpu.VMEM((CHUNK,), jnp.uint32),
            pltpu.VMEM((HR + 8, HC), jnp.int32),
            pltpu.VMEM((128,), jnp.int32),
            pltpu.SemaphoreType.DMA,
            pltpu.SemaphoreType.DMA,
        ],
    )
    return hist_top, hist_low


def _cumsum_flat(h):
    x = h
    for d in (1, 2, 4, 8, 16, 32, 64):
        x = x + jnp.concatenate(
            [jnp.zeros((HR, d), jnp.int32), x[:, : HC - d]], axis=1
        )
    rs = x[:, HC - 1 : HC]
    ro = rs
    for d in (1, 2, 4, 8, 16, 32, 64, 128, 256):
        ro = ro + jnp.concatenate(
            [jnp.zeros((d, 1), jnp.int32), ro[: HR - d, :]], axis=0
        )
    return x + (ro - rs)


def _sel_top_body(h_ref, bsel_ref, meta_ref):
    h = jnp.sum(h_ref[...], axis=0)
    cum = _cumsum_flat(h)
    row = lax.broadcasted_iota(jnp.int32, (HR, HC), 0)
    col = lax.broadcasted_iota(jnp.int32, (HR, HC), 1)
    flat = row * HC + col
    n = jnp.sum(jnp.where(flat < 32768, h, 0))
    k = lax.div(n - 1, jnp.int32(2))
    le = cum <= k
    b = jnp.sum(le.astype(jnp.int32))
    excl = jnp.max(jnp.where(le, cum, 0))
    r = k - excl
    bsel_ref[...] = jnp.zeros((8, 128), jnp.int32) + b
    riota = lax.broadcasted_iota(jnp.int32, (8, 128), 0)
    meta_ref[...] = jnp.where(riota == 0, r, n)


def _select_top(h1):
    return pl.pallas_call(
        _sel_top_body,
        in_specs=[pl.BlockSpec((NW, HR, HC), lambda: (0, 0, 0))],
        out_specs=[
            pl.BlockSpec((8, 128), lambda: (0, 0)),
            pl.BlockSpec((8, 128), lambda: (0, 0)),
        ],
        out_shape=[
            jax.ShapeDtypeStruct((8, 128), jnp.int32),
            jax.ShapeDtypeStruct((8, 128), jnp.int32),
        ],
    )(h1)


def _eq_body(h2_ref, bsel_ref, meta_ref, combo_ref, out_ref, med_sm):
    i = pl.program_id(0)
    j = pl.program_id(1)

    @pl.when((i == 0) & (j == 0))
    def _():
        h = jnp.sum(h2_ref[...], axis=0)
        cum = _cumsum_flat(h)
        r = meta_ref[0, 0]
        n = meta_ref[1, 0]
        b = bsel_ref[0, 0]
        c = jnp.sum((cum <= r).astype(jnp.int32))
        med_sm[0] = jnp.where(n == 0, jnp.int32(-1), (b << 16) | c)

    med = med_sm[0]
    bits = lax.bitcast_convert_type(
        combo_ref[...] & jnp.uint32(0x7FFFFFFF), jnp.int32
    )
    out_ref[0] = (bits == med).astype(jnp.int32).reshape(out_ref.shape[1:])


def _compare(h2, bsel, meta, combo):
    rows = 256
    grid = (B, H // rows)
    return pl.pallas_call(
        _eq_body,
        grid=grid,
        in_specs=[
            pl.BlockSpec((NW, HR, HC), lambda i, j: (0, 0, 0)),
            pl.BlockSpec((8, 128), lambda i, j: (0, 0)),
            pl.BlockSpec((8, 128), lambda i, j: (0, 0)),
            pl.BlockSpec((rows * 4, 128), lambda i, j: (i * 2 + j, 0)),
        ],
        out_specs=pl.BlockSpec((1, rows, W), lambda i, j: (i, j, 0)),
        out_shape=jax.ShapeDtypeStruct((B, H, W), jnp.int32),
        scratch_shapes=[pltpu.SMEM((1,), jnp.int32)],
    )(h2, bsel, meta, combo)


def kernel(img):
    hist_top, hist_low = _sc_kernels()
    combo = _make_combo(img)
    combo_flat = combo.reshape(N)
    h1 = hist_top(combo_flat)
    bsel, meta = _select_top(h1)
    h2 = hist_low(combo_flat, bsel)
    res = _compare(h2, bsel, meta, combo)
    return res.reshape(B, 1, H, W)


# speedup vs baseline: 34.0330x; 1.0094x over previous
import functools

import jax
import jax.numpy as jnp
from jax import lax
from jax.experimental import pallas as pl
from jax.experimental.pallas import tpu as pltpu
from jax.experimental.pallas import tpu_sc as plsc

B, C, H, W = 8, 3, 512, 512
N = B * H * W

NC, NS, L = 2, 16, 16
NW = NC * NS
PER_TILE = N // NW
CHUNK = 16384
NCHUNKS = PER_TILE // CHUNK
UNROLL = 16

HR, HC = 512, 128


def _combo_body(img_ref, out_ref):
    x = img_ref[0]
    r, g, b = x[0], x[1], x[2]
    gray = (r * jnp.float32(0.299) + g * jnp.float32(0.587)) + b * jnp.float32(0.114)
    bits = lax.bitcast_convert_type(gray, jnp.uint32)
    combo = jnp.where(
        (r + g + b) > jnp.float32(1.5), bits, bits | jnp.uint32(0x80000000)
    )
    out_ref[...] = combo.reshape(out_ref.shape)


def _make_combo(img):
    rows = 256
    grid = (B, H // rows)
    return pl.pallas_call(
        _combo_body,
        grid=grid,
        in_specs=[pl.BlockSpec((1, C, rows, W), lambda i, j: (i, 0, j, 0))],
        out_specs=pl.BlockSpec((rows * 4, 128), lambda i, j: (i * 2 + j, 0)),
        out_shape=jax.ShapeDtypeStruct((N // 128, 128), jnp.uint32),
    )(img)


def _zero_2d(hist_v, nrows):
    z = jnp.zeros((L,), jnp.int32)

    def zb(r, _):
        for j in range(HC // L):
            hist_v[r, pl.ds(j * L, L)] = z
        return 0

    lax.fori_loop(0, nrows, zb, 0)


def _hist_scan(combo_hbm, base, stages, sems, bin_fn):
    def start(ci):
        return pltpu.async_copy(
            combo_hbm.at[pl.ds(base + ci * CHUNK, CHUNK)],
            stages[ci % 2],
            sems[ci % 2],
        )

    handles = [start(0), None]
    for ci in range(NCHUNKS):
        if ci + 1 < NCHUNKS:
            handles[(ci + 1) % 2] = start(ci + 1)
        handles[ci % 2].wait()
        st = stages[ci % 2]

        @plsc.parallel_loop(0, CHUNK // L, step=1, unroll=UNROLL)
        def _(vi):
            bin_fn(st[pl.ds(vi * L, L)])


def _sc_hist_top_body(combo_hbm, out_hbm, stage0, stage1, hist_v, sem0, sem1):
    wid = lax.axis_index("s") * NC + lax.axis_index("c")
    base = wid * PER_TILE

    _zero_2d(hist_v, HR)

    ones = jnp.ones((L,), jnp.int32)

    def bin_fn(k16):
        bins = lax.convert_element_type(k16 >> 16, jnp.int32)
        plsc.addupdate_scatter(hist_v, [bins >> 7, bins & 127], ones)

    _hist_scan(combo_hbm, base, (stage0, stage1), (sem0, sem1), bin_fn)
    pltpu.sync_copy(hist_v, out_hbm.at[wid])


def _sc_hist_low_body(
    combo_hbm, bsel_hbm, out_hbm, stage0, stage1, hist_v, brow_v, sem0, sem1
):
    wid = lax.axis_index("s") * NC + lax.axis_index("c")
    base = wid * PER_TILE

    _zero_2d(hist_v, HR + 8)
    pltpu.sync_copy(bsel_hbm.at[0], brow_v)
    bv = brow_v[pl.ds(0, L)]
    btop = lax.convert_element_type(bv, jnp.uint32) << 16
    junk = jnp.full((L,), 65536, jnp.uint32)
    ones = jnp.ones((L,), jnp.int32)

    def bin_fn(k16):
        diff = k16 ^ btop
        flat = lax.convert_element_type(jnp.minimum(diff, junk), jnp.int32)
        plsc.addupdate_scatter(
            hist_v, [flat >> 7, flat & 127], ones, mask=diff < junk
        )

    _hist_scan(combo_hbm, base, (stage0, stage1), (sem0, sem1), bin_fn)
    pltpu.sync_copy(hist_v.at[pl.ds(0, HR)], out_hbm.at[wid])


@functools.cache
def _sc_kernels():
    mesh = plsc.VectorSubcoreMesh(core_axis_name="c", subcore_axis_name="s")
    params = pltpu.CompilerParams(needs_layout_passes=False)
    hist_top = pl.kernel(
        _sc_hist_top_body,
        out_type=jax.ShapeDtypeStruct((NW, HR, HC), jnp.int32),
        mesh=mesh,
        compiler_params=params,
        scratch_types=[
            pltpu.VMEM((CHUNK,), jnp.uint32),
            pltpu.VMEM((CHUNK,), jnp.uint32),
            pltpu.VMEM((HR, HC), jnp.int32),
            pltpu.SemaphoreType.DMA,
            pltpu.SemaphoreType.DMA,
        ],
    )
    hist_low = pl.kernel(
        _sc_hist_low_body,
        out_type=jax.ShapeDtypeStruct((NW, HR, HC), jnp.int32),
        mesh=mesh,
        compiler_params=params,
        scratch_types=[
            pltpu.VMEM((CHUNK,), jnp.uint32),
            pltpu.VMEM((CHUNK,), jnp.uint32),
            pltpu.VMEM((HR + 8, HC), jnp.int32),
            pltpu.VMEM((128,), jnp.int32),
            pltpu.SemaphoreType.DMA,
            pltpu.SemaphoreType.DMA,
        ],
    )
    return hist_top, hist_low


def _cumsum_flat(h):
    x = h
    for d in (1, 2, 4, 8, 16, 32, 64):
        x = x + jnp.concatenate(
            [jnp.zeros((HR, d), jnp.int32), x[:, : HC - d]], axis=1
        )
    rs = x[:, HC - 1 : HC]
    ro = rs
    for d in (1, 2, 4, 8, 16, 32, 64, 128, 256):
        ro = ro + jnp.concatenate(
            [jnp.zeros((d, 1), jnp.int32), ro[: HR - d, :]], axis=0
        )
    return x + (ro - rs)


def _sel_top_body(h_ref, bsel_ref, meta_ref):
    h = jnp.sum(h_ref[...], axis=0)
    cum = _cumsum_flat(h)
    row = lax.broadcasted_iota(jnp.int32, (HR, HC), 0)
    col = lax.broadcasted_iota(jnp.int32, (HR, HC), 1)
    flat = row * HC + col
    n = jnp.sum(jnp.where(flat < 32768, h, 0))
    k = lax.div(n - 1, jnp.int32(2))
    le = cum <= k
    b = jnp.sum(le.astype(jnp.int32))
    excl = jnp.max(jnp.where(le, cum, 0))
    r = k - excl
    bsel_ref[...] = jnp.zeros((8, 128), jnp.int32) + b
    riota = lax.broadcasted_iota(jnp.int32, (8, 128), 0)
    meta_ref[...] = jnp.where(riota == 0, r, n)


def _select_top(h1):
    return pl.pallas_call(
        _sel_top_body,
        in_specs=[pl.BlockSpec((NW, HR, HC), lambda: (0, 0, 0))],
        out_specs=[
            pl.BlockSpec((8, 128), lambda: (0, 0)),
            pl.BlockSpec((8, 128), lambda: (0, 0)),
        ],
        out_shape=[
            jax.ShapeDtypeStruct((8, 128), jnp.int32),
            jax.ShapeDtypeStruct((8, 128), jnp.int32),
        ],
    )(h1)


def _eq_body(h2_ref, bsel_ref, meta_ref, combo_ref, out_ref, med_sm):
    i = pl.program_id(0)
    j = pl.program_id(1)

    @pl.when((i == 0) & (j == 0))
    def _():
        h = jnp.sum(h2_ref[...], axis=0)
        cum = _cumsum_flat(h)
        r = meta_ref[0, 0]
        n = meta_ref[1, 0]
        b = bsel_ref[0, 0]
        c = jnp.sum((cum <= r).astype(jnp.int32))
        med_sm[0] = jnp.where(n == 0, jnp.int32(-1), (b << 16) | c)

    med = med_sm[0]
    bits = lax.bitcast_convert_type(
        combo_ref[...] & jnp.uint32(0x7FFFFFFF), jnp.int32
    )
    out_ref[0] = (bits == med).astype(jnp.int32).reshape(out_ref.shape[1:])


def _compare(h2, bsel, meta, combo):
    rows = 256
    grid = (B, H // rows)
    return pl.pallas_call(
        _eq_body,
        grid=grid,
        in_specs=[
            pl.BlockSpec((NW, HR, HC), lambda i, j: (0, 0, 0)),
            pl.BlockSpec((8, 128), lambda i, j: (0, 0)),
            pl.BlockSpec((8, 128), lambda i, j: (0, 0)),
            pl.BlockSpec((rows * 4, 128), lambda i, j: (i * 2 + j, 0)),
        ],
        out_specs=pl.BlockSpec((1, rows, W), lambda i, j: (i, j, 0)),
        out_shape=jax.ShapeDtypeStruct((B, H, W), jnp.int32),
        scratch_shapes=[pltpu.SMEM((1,), jnp.int32)],
    )(h2, bsel, meta, combo)


def kernel(img):
    hist_top, hist_low = _sc_kernels()
    combo = _make_combo(img)
    combo_flat = combo.reshape(N)
    h1 = hist_top(combo_flat)
    bsel, meta = _select_top(h1)
    h2 = hist_low(combo_flat, bsel)
    res = _compare(h2, bsel, meta, combo)
    return res.reshape(B, 1, H, W)
